# baseline (speedup 1.0000x reference)
import jax
import jax.numpy as jnp
from jax.experimental import pallas as pl

_B, _N, _C, _D = 16, 1024, 384, 2048
_KA, _KB = 20, 4
_LAMBDA = 25.0
_MU = 25.0
_NU = 1.0
_ALPHA = 0.25
_EPS = 1e-4
_BIG = 3.0e38
_IDXMASK = 1023


def _pack(dist, iota_bits):
    di = jax.lax.bitcast_convert_type(dist, jnp.int32)
    return jax.lax.bitcast_convert_type((di & ~_IDXMASK) | iota_bits,
                                        jnp.float32)


def _phase1_kernel(za_ref, zbs_ref, ga_ref, gbt_ref, k_ref, v_ref):
    za = za_ref[0]
    zbs = zbs_ref[0]
    gp = jax.lax.dot_general(za, zbs, (((1,), (1,)), ((), ())),
                             preferred_element_type=jnp.float32,
                             precision=jax.lax.Precision.DEFAULT)
    x2 = jnp.sum(za * za, axis=1, keepdims=True)
    y2 = 0.25 * jnp.sum(zbs * zbs, axis=1, keepdims=True)
    y2r = jnp.reshape(y2, (1, _N))
    d2 = (gp + y2r) + x2
    rowmin_f = jnp.maximum(jnp.min(d2, axis=1, keepdims=True), 0.0)
    colmin_f = jnp.maximum(jnp.min(d2, axis=0, keepdims=True), 0.0)

    ga = ga_ref[0]
    gbt = gbt_ref[0]
    g2p = jax.lax.dot_general(ga, gbt, (((1,), (0,)), ((), ())),
                              preferred_element_type=jnp.float32,
                              precision=jax.lax.Precision.HIGHEST)
    ga2 = jnp.sum(ga * ga, axis=1, keepdims=True)
    gb2r = 0.25 * jnp.sum(gbt * gbt, axis=0, keepdims=True)
    iota_j = jax.lax.broadcasted_iota(jnp.int32, (_N, _N), 1)
    iota_i = jax.lax.broadcasted_iota(jnp.int32, (_N, _N), 0)
    prow = _pack(g2p + gb2r, iota_j)
    prow_min = jnp.min(prow, axis=1, keepdims=True)
    pcol = _pack((g2p + gb2r) + ga2, iota_i)
    pcol_min = jnp.min(pcol, axis=0, keepdims=True)
    e_a = jnp.sum(jnp.where(prow == prow_min, d2, 0.0),
                  axis=1, keepdims=True)
    e_b = jnp.sum(jnp.where(pcol == pcol_min, d2, 0.0),
                  axis=0, keepdims=True)

    krow_g = jnp.reshape(prow_min + ga2, (1, _N))
    k_ref[0, 0:1, :] = jnp.reshape(rowmin_f, (1, _N))
    k_ref[0, 1:2, :] = colmin_f
    k_ref[0, 2:3, :] = krow_g
    k_ref[0, 3:4, :] = pcol_min
    v_ref[0, 0:1, :] = jnp.reshape(e_a, (1, _N))
    v_ref[0, 1:2, :] = e_b


def _phase2_kernel(keys_ref, vals_ref, za_ref, zb_ref, out_ref):
    keys = keys_ref[:, :, :]
    vals = vals_ref[:, :, :]
    iota_row = jax.lax.broadcasted_iota(jnp.int32, (1, 4, 1), 1)
    krow = jnp.where(iota_row < 3, _KA, _KB)

    def body(t, carry):
        ks, acc = carry
        m = jnp.min(ks, axis=2, keepdims=True)
        sel = ks == m
        gsel = jnp.sum(jnp.where(sel[:, 2:4, :], vals, 0.0),
                       axis=2, keepdims=True)
        contrib = jnp.concatenate([m[:, 0:2, :], gsel], axis=1)
        w = (t < krow).astype(jnp.float32)
        acc = acc + contrib * w
        ks = jnp.where(sel, _BIG, ks)
        return ks, acc

    acc0 = jnp.zeros((_B, 4, 1), jnp.float32)
    _, acc = jax.lax.fori_loop(0, _KA, body, (keys, acc0))

    lcoef = _LAMBDA * (1.0 - _ALPHA) * 0.5
    coef = jnp.where(iota_row < 3,
                     lcoef / (_B * _KA * _C),
                     lcoef / (_B * _KB * _C))
    local = jnp.sum(acc * coef)

    za = za_ref[:, :]
    zb = zb_ref[:, :]
    inv_g = jnp.mean((za - zb) ** 2)

    def _var_cov(x):
        mu = jnp.mean(x, axis=0, keepdims=True)
        xc = x - mu
        var = jnp.sum(xc * xc, axis=0, keepdims=True) / (_B - 1)
        std = jnp.sqrt(var + _EPS)
        vloss = jnp.mean(jnp.maximum(1.0 - std, 0.0))
        a = jax.lax.dot_general(xc, xc, (((1,), (1,)), ((), ())),
                                preferred_element_type=jnp.float32,
                                precision=jax.lax.Precision.HIGHEST)
        frob = jnp.sum(a * a) / float((_B - 1) ** 2)
        closs = (frob - jnp.sum(var * var)) / _D
        return vloss, closs

    vl_a, cl_a = _var_cov(za)
    vl_b, cl_b = _var_cov(zb)
    global_loss = (_LAMBDA * inv_g + _MU * 0.5 * (vl_a + vl_b)
                   + _NU * (cl_a + cl_b))
    total = _ALPHA * global_loss + local
    out_ref[:, :] = total * jnp.ones((1, 1), jnp.float32)


def kernel(z_a, z_b, z_a_local_features, z_b_local_features, grid_a, grid_b):
    za_l = z_a_local_features.reshape(_B, _N, _C)
    zbs_l = z_b_local_features.reshape(_B, _N, _C) * -2.0
    ga = grid_a.reshape(_B, _N, 2)
    gbt = jnp.swapaxes(grid_b.reshape(_B, _N, 2), 1, 2) * -2.0

    keys, vals = pl.pallas_call(
        _phase1_kernel,
        grid=(_B,),
        in_specs=[
            pl.BlockSpec((1, _N, _C), lambda b: (b, 0, 0)),
            pl.BlockSpec((1, _N, _C), lambda b: (b, 0, 0)),
            pl.BlockSpec((1, _N, 2), lambda b: (b, 0, 0)),
            pl.BlockSpec((1, 2, _N), lambda b: (b, 0, 0)),
        ],
        out_specs=[
            pl.BlockSpec((1, 4, _N), lambda b: (b, 0, 0)),
            pl.BlockSpec((1, 2, _N), lambda b: (b, 0, 0)),
        ],
        out_shape=[
            jax.ShapeDtypeStruct((_B, 4, _N), jnp.float32),
            jax.ShapeDtypeStruct((_B, 2, _N), jnp.float32),
        ],
    )(za_l, zbs_l, ga, gbt)

    out = pl.pallas_call(
        _phase2_kernel,
        out_shape=jax.ShapeDtypeStruct((1, 1), jnp.float32),
    )(keys, vals, z_a, z_b)
    return out.reshape(())


# speedup vs baseline: 1.0851x; 1.0851x over previous
import jax
import jax.numpy as jnp
from jax.experimental import pallas as pl
from jax.experimental.pallas import tpu as pltpu

_B, _N, _C, _D = 16, 1024, 384, 2048
_KA, _KB = 20, 4
_LAMBDA = 25.0
_MU = 25.0
_NU = 1.0
_ALPHA = 0.25
_EPS = 1e-4
_BIG = 3.0e38
_IDXMASK = 1023


def _pack(dist, iota_bits):
    di = jax.lax.bitcast_convert_type(dist, jnp.int32)
    return jax.lax.bitcast_convert_type((di & ~_IDXMASK) | iota_bits,
                                        jnp.float32)


def _unpack_idx(pmin):
    return jax.lax.bitcast_convert_type(pmin, jnp.int32) & _IDXMASK


def _kernel(za_ref, zb_ref, aga_ref, agbt_ref, ga2_ref, gb2_ref,
            zag_ref, zbg_ref, out_ref, skeys_ref, svals_ref):
    b = pl.program_id(0)
    za = za_ref[0]
    zb = zb_ref[0]
    zbs = zb * -2.0
    gp = jax.lax.dot_general(za, zbs, (((1,), (1,)), ((), ())),
                             preferred_element_type=jnp.float32,
                             precision=jax.lax.Precision.DEFAULT)
    x2 = jnp.sum(za * za, axis=1, keepdims=True)
    y2 = jnp.sum(zb * zb, axis=1, keepdims=True)
    y2r = jnp.reshape(y2, (1, _N))

    rowmin_f = jnp.maximum(x2 + jnp.min(gp + y2r, axis=1, keepdims=True), 0.0)
    colmin_f = jnp.maximum(y2r + jnp.min(gp + x2, axis=0, keepdims=True), 0.0)

    aga = aga_ref[0]
    agbt = agbt_ref[0]
    g2p = jax.lax.dot_general(aga, agbt, (((1,), (0,)), ((), ())),
                              preferred_element_type=jnp.float32,
                              precision=jax.lax.Precision.DEFAULT)
    ga2 = ga2_ref[0]
    gb2r = gb2_ref[0]
    iota_j = jax.lax.broadcasted_iota(jnp.int32, (_N, _N), 1)
    iota_i = jax.lax.broadcasted_iota(jnp.int32, (_N, _N), 0)
    prow_min = jnp.min(_pack(g2p + gb2r, iota_j), axis=1, keepdims=True)
    pcol_min = jnp.min(_pack(g2p + ga2, iota_i), axis=0, keepdims=True)
    row_arg = _unpack_idx(prow_min)
    col_arg = _unpack_idx(pcol_min)

    e_a = x2 + jnp.sum(jnp.where(iota_j == row_arg, gp + y2r, 0.0),
                       axis=1, keepdims=True)
    e_b = y2r + jnp.sum(jnp.where(iota_i == col_arg, gp + x2, 0.0),
                        axis=0, keepdims=True)

    skeys_ref[b, 0:1, :] = jnp.reshape(rowmin_f, (1, _N))
    skeys_ref[b, 1:2, :] = colmin_f
    skeys_ref[b, 2:3, :] = jnp.reshape(prow_min + ga2, (1, _N))
    skeys_ref[b, 3:4, :] = pcol_min + gb2r
    svals_ref[b, 0:1, :] = jnp.reshape(e_a, (1, _N))
    svals_ref[b, 1:2, :] = e_b

    @pl.when(b == _B - 1)
    def _tail():
        keys = skeys_ref[:, :, :]
        vals = svals_ref[:, :, :]
        iota_row = jax.lax.broadcasted_iota(jnp.int32, (1, 4, 1), 1)
        krow = jnp.where(iota_row < 3, _KA, _KB)

        def body(t, carry):
            ks, acc = carry
            m = jnp.min(ks, axis=2, keepdims=True)
            sel = ks == m
            gsel = jnp.sum(jnp.where(sel[:, 2:4, :], vals, 0.0),
                           axis=2, keepdims=True)
            contrib = jnp.concatenate([m[:, 0:2, :], gsel], axis=1)
            w = (t < krow).astype(jnp.float32)
            acc = acc + contrib * w
            ks = jnp.where(sel, _BIG, ks)
            return ks, acc

        acc0 = jnp.zeros((_B, 4, 1), jnp.float32)
        _, acc = jax.lax.fori_loop(0, _KA, body, (keys, acc0))

        lcoef = _LAMBDA * (1.0 - _ALPHA) * 0.5
        coef = jnp.where(iota_row < 3,
                         lcoef / (_B * _KA * _C),
                         lcoef / (_B * _KB * _C))
        local = jnp.sum(acc * coef)

        zag = zag_ref[:, :]
        zbg = zbg_ref[:, :]
        inv_g = jnp.mean((zag - zbg) ** 2)

        def _var_cov(x):
            mu = jnp.mean(x, axis=0, keepdims=True)
            xc = x - mu
            var = jnp.sum(xc * xc, axis=0, keepdims=True) / (_B - 1)
            std = jnp.sqrt(var + _EPS)
            vloss = jnp.mean(jnp.maximum(1.0 - std, 0.0))
            a = jax.lax.dot_general(xc, xc, (((1,), (1,)), ((), ())),
                                    preferred_element_type=jnp.float32,
                                    precision=jax.lax.Precision.HIGHEST)
            frob = jnp.sum(a * a) / float((_B - 1) ** 2)
            closs = (frob - jnp.sum(var * var)) / _D
            return vloss, closs

        vl_a, cl_a = _var_cov(zag)
        vl_b, cl_b = _var_cov(zbg)
        global_loss = (_LAMBDA * inv_g + _MU * 0.5 * (vl_a + vl_b)
                       + _NU * (cl_a + cl_b))
        total = _ALPHA * global_loss + local
        out_ref[:, :] = total * jnp.ones((1, 1), jnp.float32)


def kernel(z_a, z_b, z_a_local_features, z_b_local_features, grid_a, grid_b):
    za_l = z_a_local_features.reshape(_B, _N, _C)
    zb_l = z_b_local_features.reshape(_B, _N, _C)
    ga = grid_a.reshape(_B, _N, 2)
    gb = grid_b.reshape(_B, _N, 2)

    ga_h = ga.astype(jnp.bfloat16).astype(jnp.float32)
    ga_l = ga - ga_h
    gb_h = gb.astype(jnp.bfloat16).astype(jnp.float32)
    gb_l = gb - gb_h
    aga = jnp.concatenate([ga_h, ga_h, ga_l], axis=2)
    agb = jnp.concatenate([gb_h, gb_l, gb_h], axis=2) * -2.0
    agbt = jnp.swapaxes(agb, 1, 2)
    ga2 = jnp.sum(ga * ga, axis=2, keepdims=True)
    gb2 = jnp.sum(gb * gb, axis=2)[:, None, :]

    out = pl.pallas_call(
        _kernel,
        grid=(_B,),
        in_specs=[
            pl.BlockSpec((1, _N, _C), lambda b: (b, 0, 0)),
            pl.BlockSpec((1, _N, _C), lambda b: (b, 0, 0)),
            pl.BlockSpec((1, _N, 6), lambda b: (b, 0, 0)),
            pl.BlockSpec((1, 6, _N), lambda b: (b, 0, 0)),
            pl.BlockSpec((1, _N, 1), lambda b: (b, 0, 0)),
            pl.BlockSpec((1, 1, _N), lambda b: (b, 0, 0)),
            pl.BlockSpec((_B, _D), lambda b: (0, 0)),
            pl.BlockSpec((_B, _D), lambda b: (0, 0)),
        ],
        out_specs=pl.BlockSpec((1, 1), lambda b: (0, 0)),
        out_shape=jax.ShapeDtypeStruct((1, 1), jnp.float32),
        scratch_shapes=[
            pltpu.VMEM((_B, 4, _N), jnp.float32),
            pltpu.VMEM((_B, 2, _N), jnp.float32),
        ],
    )(za_l, zb_l, aga, agbt, ga2, gb2, z_a, z_b)
    return out.reshape(())
